# baseline (speedup 1.0000x reference)
import functools

import jax
import jax.numpy as jnp
import numpy as np
from jax import lax
from jax.experimental import pallas as pl
from jax.experimental.pallas import tpu as pltpu
from jax.experimental.pallas import tpu_sc as plsc

_L = 16
_NC = 2
_NS = 16
_NW = _NC * _NS
_N = 1201
_RPB = 8
_NCH = (_N - 1 + _L - 1) // _L
_NEG = np.float32(-3.0e38)
_POS = np.float32(3.0e38)
_BIGI = np.int32(1 << 30)


@functools.lru_cache(maxsize=None)
def _make_sc_kernel(b, sc_rows=None, row0=0):
    if sc_rows is None:
        sc_rows = b
    nblk = sc_rows // _RPB
    blk_words = _RPB * _N
    buf_words = blk_words + _L
    out_words = _RPB * _L
    mesh = plsc.VectorSubcoreMesh(core_axis_name="c", subcore_axis_name="s",
                                  num_cores=_NC, num_subcores=_NS)

    @functools.partial(
        pl.kernel,
        out_type=jax.ShapeDtypeStruct((sc_rows * _L,), jnp.float32),
        mesh=mesh,
        scratch_types=(pltpu.VMEM((buf_words,), jnp.float32),
                       pltpu.VMEM((buf_words,), jnp.float32),
                       pltpu.VMEM((out_words,), jnp.float32),
                       pltpu.VMEM((out_words,), jnp.float32),
                       pltpu.SemaphoreType.DMA,
                       pltpu.SemaphoreType.DMA,
                       pltpu.SemaphoreType.DMA,
                       pltpu.SemaphoreType.DMA),
    )
    def sc_kernel(sp_hbm, out_hbm, ibuf0, ibuf1, ost0, ost1,
                  isem0, isem1, osem0, osem1):
        wid = lax.axis_index("s") * _NC + lax.axis_index("c")
        lanes = lax.iota(jnp.int32, _L)
        nblk_w = (nblk - wid + _NW - 1) // _NW

        def in_cp(t, buf, sem):
            blk = wid + _NW * t
            return pltpu.make_async_copy(
                sp_hbm.at[pl.ds(blk * blk_words, blk_words)],
                buf.at[pl.ds(0, blk_words)], sem)

        def out_cp(t, ost, sem):
            blk = wid + _NW * t
            return pltpu.make_async_copy(
                ost, out_hbm.at[pl.ds(blk * out_words, out_words)], sem)

        def process(t, buf, ost):
            def do_row(i, _):
                b1 = jnp.where(lanes == 0, np.float32(0.0), _NEG)
                i1 = jnp.where(lanes == 0, np.int32(0), _BIGI)
                b2 = jnp.full((_L,), _NEG, jnp.float32)
                i2 = jnp.full((_L,), _BIGI, jnp.int32)
                b3 = b2
                i3 = i2
                roff = i * _N
                jv = lanes + 1
                for c in range(_NCH):
                    off = roff + c * _L
                    v = buf[pl.ds(off + 1, _L)]
                    vp = buf[pl.ds(off, _L)]
                    vn = buf[pl.ds(off + 2, _L)]
                    mask = (v - vp >= 0.0) & (vn - v <= 0.0)
                    if c == _NCH - 1:
                        mask &= jv <= _N - 2
                    pv = jnp.where(mask, v, np.float32(0.0))
                    c1 = pv > b1
                    c2 = pv > b2
                    c3 = pv > b3
                    b1, b2, b3, i1, i2, i3 = (
                        jnp.where(c1, pv, b1),
                        jnp.where(c1, b1, jnp.where(c2, pv, b2)),
                        jnp.where(c2, b2, jnp.where(c3, pv, b3)),
                        jnp.where(c1, jv, i1),
                        jnp.where(c1, i1, jnp.where(c2, jv, i2)),
                        jnp.where(c2, i2, jnp.where(c3, jv, i3)),
                    )
                    if c != _NCH - 1:
                        jv = jv + _L

                def _bfly(x, op):
                    for s in (8, 4, 2, 1):
                        x = op(x, jnp.take(x, lanes ^ s))
                    return x

                ms = []
                gis = []
                for _t in range(3):
                    m = _bfly(b1, jnp.maximum)
                    gi = _bfly(jnp.where(b1 == m, i1, _BIGI), jnp.minimum)
                    ms.append(m)
                    gis.append(gi)
                    lm = (b1 == m) & (i1 == gi)
                    b1 = jnp.where(lm, b2, b1)
                    i1 = jnp.where(lm, i2, i1)
                    b2 = jnp.where(lm, b3, b2)
                    i2 = jnp.where(lm, i3, i2)
                    b3 = jnp.where(lm, _NEG, b3)
                    i3 = jnp.where(lm, _BIGI, i3)
                s0 = jnp.minimum(jnp.minimum(gis[0], gis[1]), gis[2])
                s2 = jnp.maximum(jnp.maximum(gis[0], gis[1]), gis[2])
                s1 = gis[0] + gis[1] + gis[2] - s0 - s2
                idxv = jnp.where(lanes == 0, s0, jnp.where(lanes == 1, s1, s2))
                tv = idxv.astype(jnp.float32) * 0.1 - 60.0
                mv = jnp.where(lanes == 3, ms[0],
                               jnp.where(lanes == 4, ms[1], ms[2]))
                sv = jnp.where(mv != 0.0, np.float32(1.0), np.float32(0.0))
                ost[pl.ds(i * _L, _L)] = jnp.where(lanes < 3, tv, sv)
                return 0

            lax.fori_loop(0, _RPB, do_row, 0)

        in_cp(0, ibuf0, isem0).start()
        nt2 = (nblk_w + 1) // 2

        @pl.loop(0, nt2)
        def _t2loop(t2):
            for p in (0, 1):
                buf, ost, isem, osem = ((ibuf0, ost0, isem0, osem0),
                                        (ibuf1, ost1, isem1, osem1))[p]
                nbuf, nisem = ((ibuf1, isem1), (ibuf0, isem0))[p]
                t = 2 * t2 + p

                @pl.when(t < nblk_w)
                def _():
                    @pl.when(t + 1 < nblk_w)
                    def _():
                        in_cp(t + 1, nbuf, nisem).start()

                    in_cp(t, buf, isem).wait()

                    @pl.when(t >= 2)
                    def _():
                        out_cp(t, ost, osem).wait()

                    process(t, buf, ost)
                    out_cp(t, ost, osem).start()

        out_cp(0, ost0, osem0).wait()
        out_cp(0, ost1, osem1).wait()

    return sc_kernel


_TBLK = 256


@functools.lru_cache(maxsize=None)
def _make_tc_kernel(rows):
    grid = rows // _TBLK

    def body(x_ref, g_ref, o_ref):
        x = x_ref[...]
        gr = jnp.broadcast_to(g_ref[...], (_TBLK, _N))
        d = x[:, 1:] - x[:, :-1]
        neg1 = jnp.full((_TBLK, 1), np.float32(-1.0))
        pos1 = jnp.full((_TBLK, 1), np.float32(1.0))
        dl = jnp.concatenate([neg1, d], axis=1)
        dr = jnp.concatenate([d, pos1], axis=1)
        pv = jnp.where((dl >= 0.0) & (dr <= 0.0), x, np.float32(0.0))
        ms = []
        gvs = []
        for _t in range(3):
            m = jnp.max(pv, axis=1, keepdims=True)
            gv = jnp.min(jnp.where(pv == m, gr, _POS), axis=1, keepdims=True)
            ms.append(m)
            gvs.append(gv)
            pv = jnp.where(gr == gv, _NEG, pv)
        g1, g2, g3 = gvs
        s0 = jnp.minimum(jnp.minimum(g1, g2), g3)
        s2 = jnp.maximum(jnp.maximum(g1, g2), g3)
        s1 = jnp.maximum(jnp.minimum(g1, g2),
                         jnp.minimum(jnp.maximum(g1, g2), g3))
        lane16 = lax.broadcasted_iota(jnp.int32, (_TBLK, _L), 1)
        tv = jnp.where(lane16 == 0, s0, jnp.where(lane16 == 1, s1, s2))
        mv = jnp.where(lane16 == 3, ms[0], jnp.where(lane16 == 4, ms[1], ms[2]))
        sv = jnp.where(mv != 0.0, np.float32(1.0), np.float32(0.0))
        o_ref[...] = jnp.where(lane16 < 3, tv, sv)

    return pl.pallas_call(
        body,
        grid=(grid,),
        in_specs=[pl.BlockSpec((_TBLK, _N), lambda i: (i, 0)),
                  pl.BlockSpec((1, _N), lambda i: (0, 0))],
        out_specs=pl.BlockSpec((_TBLK, _L), lambda i: (i, 0)),
        out_shape=jax.ShapeDtypeStruct((rows, _L), jnp.float32),
    )


_SC_ROWS = 6688


def kernel(sp_batch, k):
    b, n = sp_batch.shape
    assert n == _N
    tc_rows = b - _SC_ROWS
    assert tc_rows % _TBLK == 0 and _SC_ROWS % _RPB == 0
    grid_row = (jnp.arange(_N, dtype=jnp.float32) * 0.1 - 60.0).reshape(1, _N)
    one = (k - k + 1).astype(jnp.float32)
    sp_tail = (sp_batch[tc_rows:] * one).reshape(_SC_ROWS * _N)
    out_sc = _make_sc_kernel(_SC_ROWS)(sp_tail).reshape(_SC_ROWS, _L)
    out_tc = _make_tc_kernel(tc_rows)(sp_batch, grid_row)
    out = jnp.concatenate([out_tc, out_sc], axis=0)
    theta = out[:, :3]
    succ = jnp.take(out[:, 3:6], k - 1, axis=-1).astype(jnp.bool_)
    return (succ, theta)


# speedup vs baseline: 1.2081x; 1.2081x over previous
import functools

import jax
import jax.numpy as jnp
import numpy as np
from jax import lax
from jax.experimental import pallas as pl
from jax.experimental.pallas import tpu as pltpu
from jax.experimental.pallas import tpu_sc as plsc

_L = 16
_NC = 2
_NS = 16
_NW = _NC * _NS
_N = 1201
_RPB = 8
_NCH = (_N - 1 + _L - 1) // _L
_NEG = np.float32(-3.0e38)
_POS = np.float32(3.0e38)
_BIGI = np.int32(1 << 30)


@functools.lru_cache(maxsize=None)
def _make_sc_kernel(b, sc_rows=None, row0=0):
    if sc_rows is None:
        sc_rows = b
    nblk = sc_rows // _RPB
    blk_words = _RPB * _N
    buf_words = blk_words + _L
    out_words = _RPB * _L
    mesh = plsc.VectorSubcoreMesh(core_axis_name="c", subcore_axis_name="s",
                                  num_cores=_NC, num_subcores=_NS)

    @functools.partial(
        pl.kernel,
        out_type=jax.ShapeDtypeStruct((sc_rows * _L,), jnp.float32),
        mesh=mesh,
        scratch_types=(pltpu.VMEM((buf_words,), jnp.float32),
                       pltpu.VMEM((buf_words,), jnp.float32),
                       pltpu.VMEM((out_words,), jnp.float32),
                       pltpu.VMEM((out_words,), jnp.float32),
                       pltpu.SemaphoreType.DMA,
                       pltpu.SemaphoreType.DMA,
                       pltpu.SemaphoreType.DMA,
                       pltpu.SemaphoreType.DMA),
    )
    def sc_kernel(sp_hbm, out_hbm, ibuf0, ibuf1, ost0, ost1,
                  isem0, isem1, osem0, osem1):
        wid = lax.axis_index("s") * _NC + lax.axis_index("c")
        lanes = lax.iota(jnp.int32, _L)
        nblk_w = (nblk - wid + _NW - 1) // _NW

        def in_cp(t, buf, sem):
            blk = wid + _NW * t
            return pltpu.make_async_copy(
                sp_hbm.at[pl.ds(row0 * _N + blk * blk_words, blk_words)],
                buf.at[pl.ds(0, blk_words)], sem)

        def out_cp(t, ost, sem):
            blk = wid + _NW * t
            return pltpu.make_async_copy(
                ost, out_hbm.at[pl.ds(blk * out_words, out_words)], sem)

        def process(t, buf, ost):
            def do_row(i, _):
                b1 = jnp.where(lanes == 0, np.float32(0.0), _NEG)
                i1 = jnp.where(lanes == 0, np.int32(0), _BIGI)
                b2 = jnp.full((_L,), _NEG, jnp.float32)
                i2 = jnp.full((_L,), _BIGI, jnp.int32)
                b3 = b2
                i3 = i2
                roff = i * _N
                jv = lanes + 1
                for c in range(_NCH):
                    off = roff + c * _L
                    v = buf[pl.ds(off + 1, _L)]
                    vp = buf[pl.ds(off, _L)]
                    vn = buf[pl.ds(off + 2, _L)]
                    mask = (v - vp >= 0.0) & (vn - v <= 0.0)
                    if c == _NCH - 1:
                        mask &= jv <= _N - 2
                    pv = jnp.where(mask, v, np.float32(0.0))
                    c1 = pv > b1
                    c2 = pv > b2
                    c3 = pv > b3
                    b1, b2, b3, i1, i2, i3 = (
                        jnp.where(c1, pv, b1),
                        jnp.where(c1, b1, jnp.where(c2, pv, b2)),
                        jnp.where(c2, b2, jnp.where(c3, pv, b3)),
                        jnp.where(c1, jv, i1),
                        jnp.where(c1, i1, jnp.where(c2, jv, i2)),
                        jnp.where(c2, i2, jnp.where(c3, jv, i3)),
                    )
                    if c != _NCH - 1:
                        jv = jv + _L

                def _bfly(x, op):
                    for s in (8, 4, 2, 1):
                        x = op(x, jnp.take(x, lanes ^ s))
                    return x

                ms = []
                gis = []
                for _t in range(3):
                    m = _bfly(b1, jnp.maximum)
                    gi = _bfly(jnp.where(b1 == m, i1, _BIGI), jnp.minimum)
                    ms.append(m)
                    gis.append(gi)
                    lm = (b1 == m) & (i1 == gi)
                    b1 = jnp.where(lm, b2, b1)
                    i1 = jnp.where(lm, i2, i1)
                    b2 = jnp.where(lm, b3, b2)
                    i2 = jnp.where(lm, i3, i2)
                    b3 = jnp.where(lm, _NEG, b3)
                    i3 = jnp.where(lm, _BIGI, i3)
                s0 = jnp.minimum(jnp.minimum(gis[0], gis[1]), gis[2])
                s2 = jnp.maximum(jnp.maximum(gis[0], gis[1]), gis[2])
                s1 = gis[0] + gis[1] + gis[2] - s0 - s2
                idxv = jnp.where(lanes == 0, s0, jnp.where(lanes == 1, s1, s2))
                tv = idxv.astype(jnp.float32) * 0.1 - 60.0
                mv = jnp.where(lanes == 3, ms[0],
                               jnp.where(lanes == 4, ms[1], ms[2]))
                sv = jnp.where(mv != 0.0, np.float32(1.0), np.float32(0.0))
                ost[pl.ds(i * _L, _L)] = jnp.where(lanes < 3, tv, sv)
                return 0

            lax.fori_loop(0, _RPB, do_row, 0)

        in_cp(0, ibuf0, isem0).start()
        nt2 = (nblk_w + 1) // 2

        @pl.loop(0, nt2)
        def _t2loop(t2):
            for p in (0, 1):
                buf, ost, isem, osem = ((ibuf0, ost0, isem0, osem0),
                                        (ibuf1, ost1, isem1, osem1))[p]
                nbuf, nisem = ((ibuf1, isem1), (ibuf0, isem0))[p]
                t = 2 * t2 + p

                @pl.when(t < nblk_w)
                def _():
                    @pl.when(t + 1 < nblk_w)
                    def _():
                        in_cp(t + 1, nbuf, nisem).start()

                    in_cp(t, buf, isem).wait()

                    @pl.when(t >= 2)
                    def _():
                        out_cp(t, ost, osem).wait()

                    process(t, buf, ost)
                    out_cp(t, ost, osem).start()

        out_cp(0, ost0, osem0).wait()
        out_cp(0, ost1, osem1).wait()

    return sc_kernel


_TBLK = 256


@functools.lru_cache(maxsize=None)
def _make_tc_kernel(rows):
    grid = rows // _TBLK

    def body(x_ref, g_ref, o_ref):
        x = x_ref[...]
        gr = jnp.broadcast_to(g_ref[...], (_TBLK, _N))
        d = x[:, 1:] - x[:, :-1]
        neg1 = jnp.full((_TBLK, 1), np.float32(-1.0))
        pos1 = jnp.full((_TBLK, 1), np.float32(1.0))
        dl = jnp.concatenate([neg1, d], axis=1)
        dr = jnp.concatenate([d, pos1], axis=1)
        pv = jnp.where((dl >= 0.0) & (dr <= 0.0), x, np.float32(0.0))
        ms = []
        gvs = []
        for _t in range(3):
            m = jnp.max(pv, axis=1, keepdims=True)
            gv = jnp.min(jnp.where(pv == m, gr, _POS), axis=1, keepdims=True)
            ms.append(m)
            gvs.append(gv)
            pv = jnp.where(gr == gv, _NEG, pv)
        g1, g2, g3 = gvs
        s0 = jnp.minimum(jnp.minimum(g1, g2), g3)
        s2 = jnp.maximum(jnp.maximum(g1, g2), g3)
        s1 = jnp.maximum(jnp.minimum(g1, g2),
                         jnp.minimum(jnp.maximum(g1, g2), g3))
        lane16 = lax.broadcasted_iota(jnp.int32, (_TBLK, _L), 1)
        tv = jnp.where(lane16 == 0, s0, jnp.where(lane16 == 1, s1, s2))
        mv = jnp.where(lane16 == 3, ms[0], jnp.where(lane16 == 4, ms[1], ms[2]))
        sv = jnp.where(mv != 0.0, np.float32(1.0), np.float32(0.0))
        o_ref[...] = jnp.where(lane16 < 3, tv, sv)

    return pl.pallas_call(
        body,
        grid=(grid,),
        in_specs=[pl.BlockSpec((_TBLK, _N), lambda i: (i, 0)),
                  pl.BlockSpec((1, _N), lambda i: (0, 0))],
        out_specs=pl.BlockSpec((_TBLK, _L), lambda i: (i, 0)),
        out_shape=jax.ShapeDtypeStruct((rows, _L), jnp.float32),
    )


_SC_ROWS = 6688


def kernel(sp_batch, k):
    b, n = sp_batch.shape
    assert n == _N
    tc_rows = b - _SC_ROWS
    assert tc_rows % _TBLK == 0 and _SC_ROWS % _RPB == 0
    grid_row = (jnp.arange(_N, dtype=jnp.float32) * 0.1 - 60.0).reshape(1, _N)
    out_sc = _make_sc_kernel(b, _SC_ROWS, tc_rows)(
        sp_batch.reshape(-1)).reshape(_SC_ROWS, _L)
    out_tc = _make_tc_kernel(tc_rows)(sp_batch, grid_row)
    out = jnp.concatenate([out_tc, out_sc], axis=0)
    theta = out[:, :3]
    succ = jnp.take(out[:, 3:6], k - 1, axis=-1).astype(jnp.bool_)
    return (succ, theta)
